# baseline (speedup 1.0000x reference)
import functools

import jax
import jax.numpy as jnp
from jax import lax
from jax.experimental import pallas as pl
from jax.experimental.pallas import tpu as pltpu
from jax.experimental.pallas import tpu_sc as plsc

N = 5000
U = 10000
NP = 5120
UPAD = 10240
NEG = 0.01
NC, NS, LANES = 2, 16, 16
K = 128

E1, E1P = 160000, 163840
E2, E2P = 320000, 321536
ZROW = 5002
C1_CHUNKS = E1P // (NC * NS) // K
C2_CHUNKS = E2P // NS // K

_mesh = plsc.VectorSubcoreMesh(core_axis_name="c", subcore_axis_name="s")


def _zero16f():
    return jnp.zeros((LANES,), jnp.float32)


def _proj_body(x_ref, w_ref, b_ref, o_ref):
    o_ref[...] = (
        jnp.dot(x_ref[...], w_ref[...], preferred_element_type=jnp.float32)
        + b_ref[...]
    )


def _proj2_body(h_ref, w_ref, b_ref, o_ref):
    h = h_ref[0, :, :64] + h_ref[1, :, :64]
    o_ref[...] = (
        jnp.dot(h, w_ref[...], preferred_element_type=jnp.float32) + b_ref[...]
    )


def _combine_body(a_ref, o_ref):
    z = a_ref[0, :U, :] + a_ref[1, :U, :]
    o_ref[...] = jnp.maximum(z, NEG * z)


CPR = NP // 2

@functools.partial(
    pl.kernel,
    out_type=(
        jax.ShapeDtypeStruct((NC, NP, 128), jnp.float32),
        jax.ShapeDtypeStruct((UPAD,), jnp.int32),
        jax.ShapeDtypeStruct((UPAD,), jnp.float32),
    ),
    mesh=_mesh,
    scratch_types=[
        pltpu.VMEM((K,), jnp.int32),
        pltpu.VMEM((K,), jnp.int32),
        pltpu.VMEM((K,), jnp.int32),
        pltpu.VMEM((K,), jnp.int32),
        pltpu.VMEM((K, 128), jnp.float32),
        pltpu.VMEM((K, 128), jnp.float32),
        pltpu.VMEM((K, 128), jnp.float32),
        pltpu.VMEM((CPR // NS, 128), jnp.float32),
        pltpu.VMEM((UPAD,), jnp.int32),
        pltpu.VMEM((UPAD,), jnp.float32),
        pltpu.VMEM((NP,), jnp.int32),
        pltpu.VMEM((NP,), jnp.float32),
        pltpu.VMEM_SHARED((CPR, 128), jnp.float32),
        pltpu.SemaphoreType.DMA,
        pltpu.SemaphoreType.DMA,
    ],
    compiler_params=pltpu.CompilerParams(needs_layout_passes=False),
)
def _sc_coarse(tq1_hbm, src_hbm, dst_hbm, mid_hbm, wts_hbm,
               h_out, inv_out, w_out,
               src_v, dst_v, gs_v, gd_v, trows, qrows, mbuf, zbuf, inv_v,
               w_v, mid_v, wts_v, acc, sem_t, sem_q):
    c = lax.axis_index("c")
    s = lax.axis_index("s")
    rows = CPR // NS

    def zrow(i, carry):
        for d in range(8):
            zbuf[i, pl.ds(d * LANES, LANES)] = _zero16f()
        return carry

    lax.fori_loop(0, rows, zrow, 0)

    def zmsg(i, carry):
        for d in range(4):
            mbuf[i, pl.ds(64 + d * LANES, LANES)] = _zero16f()
        return carry

    lax.fori_loop(0, K, zmsg, 0)

    base0 = (c * NS + s) * (C1_CHUNKS * K)

    def chunk(g, lo):
        b = base0 + g * K
        pltpu.sync_copy(src_hbm.at[pl.ds(b, K)], src_v)
        pltpu.sync_copy(dst_hbm.at[pl.ds(b, K)], dst_v)
        for grp in range(K // LANES):
            sl = pl.ds(grp * LANES, LANES)
            sv = src_v[sl]
            dv = dst_v[sl]
            dl = dv - lo
            inb = jnp.logical_and(dl >= 0, dl < CPR)
            dst_v[sl] = jnp.where(inb, dl, 0)
            gd_v[sl] = jnp.where(inb, dv, ZROW)
            gs_v[sl] = jnp.where(inb, sv, ZROW)
        ct = pltpu.async_copy(tq1_hbm.at[gd_v], trows, sem_t)
        cq = pltpu.async_copy(tq1_hbm.at[gs_v], qrows, sem_q)
        ct.wait()
        cq.wait()

        def row(i, rc):
            rsp = jnp.zeros((LANES,), jnp.int32) + i
            for d in range(4):
                ci = d * LANES + lax.iota(jnp.int32, LANES)
                z = (plsc.load_gather(trows, [rsp, ci])
                     + plsc.load_gather(qrows, [rsp, 64 + ci]))
                plsc.store_scatter(mbuf, [rsp, ci], jnp.maximum(z, NEG * z))
            return rc

        lax.fori_loop(0, K, row, 0)
        pltpu.sync_copy(mbuf, acc.at[dst_v], add=True)
        return lo

    for p in range(2):
        lo = p * CPR
        pltpu.sync_copy(zbuf, acc.at[pl.ds(s * rows, rows)])
        plsc.subcore_barrier()
        lax.fori_loop(0, C1_CHUNKS, chunk, lo)
        plsc.subcore_barrier()
        pltpu.sync_copy(acc.at[pl.ds(s * rows, rows)],
                        h_out.at[c, pl.ds(lo + s * rows, rows)])

    @pl.when(jnp.logical_and(c == 1, s == 0))
    def _aux():
        pltpu.sync_copy(mid_hbm, mid_v)
        pltpu.sync_copy(wts_hbm, wts_v)

        def ini(i, carry):
            sl = pl.ds(i * LANES, LANES)
            inv_v[sl] = jnp.full((LANES,), N, jnp.int32)
            w_v[sl] = _zero16f()
            return carry

        lax.fori_loop(0, UPAD // LANES, ini, 0)

        def sca(i, carry):
            sl = pl.ds(i * LANES, LANES)
            idx = mid_v[sl]
            plsc.store_scatter(inv_v, [idx],
                               i * LANES + lax.iota(jnp.int32, LANES))
            plsc.store_scatter(w_v, [idx], wts_v[sl])
            return carry

        lax.fori_loop(0, NP // LANES, sca, 0)
        pltpu.sync_copy(inv_v, inv_out)
        pltpu.sync_copy(w_v, w_out)


PR = UPAD // 2


@functools.partial(
    pl.kernel,
    out_type=jax.ShapeDtypeStruct((NC, UPAD, 128), jnp.float32),
    mesh=_mesh,
    scratch_types=[
        pltpu.VMEM((K,), jnp.int32),
        pltpu.VMEM((K,), jnp.int32),
        pltpu.VMEM((K,), jnp.int32),
        pltpu.VMEM((K,), jnp.int32),
        pltpu.VMEM((K, 128), jnp.float32),
        pltpu.VMEM((K, 128), jnp.float32),
        pltpu.VMEM((PR // NS, 128), jnp.float32),
        pltpu.VMEM((UPAD,), jnp.int32),
        pltpu.VMEM_SHARED((PR, 128), jnp.float32),
        pltpu.SemaphoreType.DMA,
        pltpu.SemaphoreType.DMA,
    ],
    compiler_params=pltpu.CompilerParams(needs_layout_passes=False),
)
def _sc_fine(ts_hbm, qs_hbm, t2_hbm, q2_hbm, src_hbm, dst_hbm, inv_hbm,
             out_hbm,
             src_v, dst_v, s2_v, d2_v, trows, qrows, zbuf, inv_v,
             acc, sem_t, sem_q):
    c = lax.axis_index("c")
    s = lax.axis_index("s")
    rows = PR // NS

    def zrow(i, carry):
        for d in range(8):
            zbuf[i, pl.ds(d * LANES, LANES)] = _zero16f()
        return carry

    lax.fori_loop(0, rows, zrow, 0)
    pltpu.sync_copy(inv_hbm, inv_v)

    base0 = s * (C2_CHUNKS * K)

    def run(t_hbm, q_hbm, lo):
        def chunk(g, carry):
            b = base0 + g * K
            pltpu.sync_copy(src_hbm.at[pl.ds(b, K)], src_v)
            pltpu.sync_copy(dst_hbm.at[pl.ds(b, K)], dst_v)
            for grp in range(K // LANES):
                sl = pl.ds(grp * LANES, LANES)
                sv = src_v[sl]
                dv = dst_v[sl]
                s2 = plsc.load_gather(inv_v, [sv])
                d2 = plsc.load_gather(inv_v, [dv])
                dl = dv - lo
                inb = jnp.logical_and(dl >= 0, dl < PR)
                dst_v[sl] = jnp.where(inb, dl, 0)
                s2_v[sl] = jnp.where(inb, s2, ZROW)
                d2_v[sl] = jnp.where(inb, d2, ZROW)
            ct = pltpu.async_copy(t_hbm.at[d2_v], trows, sem_t)
            cq = pltpu.async_copy(q_hbm.at[s2_v], qrows, sem_q)
            ct.wait()
            cq.wait()

            def row(i, rc):
                rsp = jnp.zeros((LANES,), jnp.int32) + i
                for d in range(8):
                    ci = d * LANES + lax.iota(jnp.int32, LANES)
                    z = (plsc.load_gather(trows, [rsp, ci])
                         + plsc.load_gather(qrows, [rsp, ci]))
                    plsc.store_scatter(trows, [rsp, ci],
                                       jnp.maximum(z, NEG * z))
                return rc

            lax.fori_loop(0, K, row, 0)
            pltpu.sync_copy(trows, acc.at[dst_v], add=True)
            return carry

        lax.fori_loop(0, C2_CHUNKS, chunk, 0)

    for p in range(2):
        lo = p * PR
        pltpu.sync_copy(zbuf, acc.at[pl.ds(s * rows, rows)])
        plsc.subcore_barrier()

        @pl.when(c == 0)
        def _skip():
            run(ts_hbm, qs_hbm, lo)

        @pl.when(c == 1)
        def _main():
            run(t2_hbm, q2_hbm, lo)

        plsc.subcore_barrier()
        pltpu.sync_copy(acc.at[pl.ds(s * rows, rows)],
                        out_hbm.at[c, pl.ds(lo + s * rows, rows)])


def kernel(x, weights, edge_index_small, m_g, m_id, W1, b1, W2, b2, Ws, bs):
    f32, i32 = jnp.float32, jnp.int32

    x_pad = jnp.zeros((NP, 128), f32).at[:N].set(x)
    W1a, W1b = W1[:128], W1[128:]
    Wsa, Wsb = Ws[:128], Ws[128:]
    Wcat = jnp.concatenate([W1a - W1b, W1b, Wsa - Wsb, Wsb], axis=1)
    bcat = jnp.concatenate(
        [b1, jnp.zeros((64,), f32), bs, jnp.zeros((128,), f32)])[None, :]

    proj = pl.pallas_call(
        _proj_body,
        out_shape=jax.ShapeDtypeStruct((NP, 384), f32),
    )(x_pad, Wcat, bcat)
    zrow = jnp.zeros((128,), f32)
    tq1 = jnp.asarray(proj[:, 0:128]).at[ZROW].set(zrow)
    ts = jnp.asarray(proj[:, 128:256]).at[ZROW].set(zrow)
    qs = jnp.asarray(proj[:, 256:384]).at[ZROW].set(zrow)

    pad1 = jnp.full((E1P - E1,), 5001, i32)
    src1 = jnp.concatenate([edge_index_small[0], pad1])
    dst1 = jnp.concatenate([edge_index_small[1], pad1])
    mid_pad = jnp.concatenate([m_id, jnp.full((NP - N,), UPAD - 8, i32)])
    w_pad = jnp.concatenate([weights[:, 0], jnp.zeros((NP - N,), f32)])

    h_parts, inv_map, w_flat = _sc_coarse(tq1, src1, dst1, mid_pad, w_pad)

    W2a, W2b = W2[:64], W2[64:]
    W2cat = jnp.concatenate([W2a - W2b, W2b], axis=1)
    b2cat = jnp.concatenate([b2, jnp.zeros((128,), f32)])[None, :]
    proj2 = pl.pallas_call(
        _proj2_body,
        out_shape=jax.ShapeDtypeStruct((NP, 256), f32),
    )(h_parts, W2cat, b2cat)
    t2 = jnp.asarray(proj2[:, 0:128]).at[ZROW].set(zrow)
    q2 = jnp.asarray(proj2[:, 128:256]).at[ZROW].set(zrow)

    pad2 = jnp.full((E2P - E2,), U, i32)
    src2 = jnp.concatenate([m_g[0], pad2])
    dst2 = jnp.concatenate([m_g[1], pad2])

    if False:
        s2j = inv_map[src2[:E2]]
        d2j = inv_map[dst2[:E2]]
        zs = ts[d2j] + qs[s2j]
        zm = t2[d2j] + q2[s2j]
        dstj = dst2[:E2]
        skipj = jax.ops.segment_sum(jnp.maximum(zs, NEG * zs), dstj,
                                    num_segments=UPAD)
        h2j = jax.ops.segment_sum(jnp.maximum(zm, NEG * zm), dstj,
                                  num_segments=UPAD)
        acc2 = jnp.stack([skipj, h2j])
    else:
        acc2 = _sc_fine(ts, qs, t2, q2, src2, dst2, inv_map)

    out = pl.pallas_call(
        _combine_body,
        out_shape=jax.ShapeDtypeStruct((U, 128), f32),
    )(acc2)
    w_up = w_flat[:U, None]
    return out, w_up


# speedup vs baseline: 1.0007x; 1.0007x over previous
import functools

import jax
import jax.numpy as jnp
from jax import lax
from jax.experimental import pallas as pl
from jax.experimental.pallas import tpu as pltpu
from jax.experimental.pallas import tpu_sc as plsc

N = 5000
U = 10000
NP = 5120
UPAD = 10240
NEG = 0.01
NC, NS, LANES = 2, 16, 16
K = 128

E1, E1P = 160000, 163840
E2, E2P = 320000, 321536
ZROW = 5002
C1_CHUNKS = E1P // (NC * NS) // K
C2_CHUNKS = E2P // NS // K

_mesh = plsc.VectorSubcoreMesh(core_axis_name="c", subcore_axis_name="s")


def _zero16f():
    return jnp.zeros((LANES,), jnp.float32)


def _proj_body(x_ref, w_ref, b_ref, o_ref):
    o_ref[...] = (
        jnp.dot(x_ref[...], w_ref[...], preferred_element_type=jnp.float32)
        + b_ref[...]
    )


def _proj2_body(h_ref, w_ref, b_ref, o_ref):
    h = h_ref[0, :, :64] + h_ref[1, :, :64]
    o_ref[...] = (
        jnp.dot(h, w_ref[...], preferred_element_type=jnp.float32) + b_ref[...]
    )


def _combine_body(a_ref, o_ref):
    z = a_ref[0, :U, :] + a_ref[1, :U, :]
    o_ref[...] = jnp.maximum(z, NEG * z)


CPR = NP // 2

@functools.partial(
    pl.kernel,
    out_type=(
        jax.ShapeDtypeStruct((NC, NP, 128), jnp.float32),
        jax.ShapeDtypeStruct((UPAD,), jnp.int32),
        jax.ShapeDtypeStruct((UPAD,), jnp.float32),
    ),
    mesh=_mesh,
    scratch_types=[
        pltpu.VMEM((K,), jnp.int32),
        pltpu.VMEM((K,), jnp.int32),
        pltpu.VMEM((K,), jnp.int32),
        pltpu.VMEM((K,), jnp.int32),
        pltpu.VMEM((K, 128), jnp.float32),
        pltpu.VMEM((K, 128), jnp.float32),
        pltpu.VMEM((K, 128), jnp.float32),
        pltpu.VMEM((CPR // NS, 128), jnp.float32),
        pltpu.VMEM((UPAD,), jnp.int32),
        pltpu.VMEM((UPAD,), jnp.float32),
        pltpu.VMEM((NP,), jnp.int32),
        pltpu.VMEM((NP,), jnp.float32),
        pltpu.VMEM_SHARED((CPR, 128), jnp.float32),
        pltpu.SemaphoreType.DMA,
        pltpu.SemaphoreType.DMA,
    ],
    compiler_params=pltpu.CompilerParams(needs_layout_passes=False),
)
def _sc_coarse(tq1_hbm, src_hbm, dst_hbm, mid_hbm, wts_hbm,
               h_out, inv_out, w_out,
               src_v, dst_v, gs_v, gd_v, trows, qrows, mbuf, zbuf, inv_v,
               w_v, mid_v, wts_v, acc, sem_t, sem_q):
    c = lax.axis_index("c")
    s = lax.axis_index("s")
    rows = CPR // NS

    def zrow(i, carry):
        for d in range(8):
            zbuf[i, pl.ds(d * LANES, LANES)] = _zero16f()
        return carry

    lax.fori_loop(0, rows, zrow, 0)

    def zmsg(i, carry):
        for d in range(4):
            mbuf[i, pl.ds(64 + d * LANES, LANES)] = _zero16f()
        return carry

    lax.fori_loop(0, K, zmsg, 0)

    base0 = (c * NS + s) * (C1_CHUNKS * K)

    def chunk(g, lo):
        b = base0 + g * K
        pltpu.sync_copy(src_hbm.at[pl.ds(b, K)], src_v)
        pltpu.sync_copy(dst_hbm.at[pl.ds(b, K)], dst_v)
        for grp in range(K // LANES):
            sl = pl.ds(grp * LANES, LANES)
            sv = src_v[sl]
            dv = dst_v[sl]
            dl = dv - lo
            inb = jnp.logical_and(dl >= 0, dl < CPR)
            dst_v[sl] = jnp.where(inb, dl, 0)
            gd_v[sl] = jnp.where(inb, dv, ZROW)
            gs_v[sl] = jnp.where(inb, sv, ZROW)
        ct = pltpu.async_copy(tq1_hbm.at[gd_v], trows, sem_t)
        cq = pltpu.async_copy(tq1_hbm.at[gs_v], qrows, sem_q)
        ct.wait()
        cq.wait()

        def row(i, rc):
            rsp = jnp.zeros((LANES,), jnp.int32) + i
            for d in range(4):
                ci = d * LANES + lax.iota(jnp.int32, LANES)
                z = (plsc.load_gather(trows, [rsp, ci])
                     + plsc.load_gather(qrows, [rsp, 64 + ci]))
                plsc.store_scatter(mbuf, [rsp, ci], jnp.maximum(z, NEG * z))
            return rc

        lax.fori_loop(0, K, row, 0)
        pltpu.sync_copy(mbuf, acc.at[dst_v], add=True)
        return lo

    for p in range(2):
        lo = p * CPR
        pltpu.sync_copy(zbuf, acc.at[pl.ds(s * rows, rows)])
        plsc.subcore_barrier()
        lax.fori_loop(0, C1_CHUNKS, chunk, lo)
        plsc.subcore_barrier()
        pltpu.sync_copy(acc.at[pl.ds(s * rows, rows)],
                        h_out.at[c, pl.ds(lo + s * rows, rows)])

    @pl.when(jnp.logical_and(c == 1, s == 0))
    def _aux():
        pltpu.sync_copy(mid_hbm, mid_v)
        pltpu.sync_copy(wts_hbm, wts_v)

        def ini(i, carry):
            sl = pl.ds(i * LANES, LANES)
            inv_v[sl] = jnp.full((LANES,), N, jnp.int32)
            w_v[sl] = _zero16f()
            return carry

        lax.fori_loop(0, UPAD // LANES, ini, 0)

        def sca(i, carry):
            sl = pl.ds(i * LANES, LANES)
            idx = mid_v[sl]
            plsc.store_scatter(inv_v, [idx],
                               i * LANES + lax.iota(jnp.int32, LANES))
            plsc.store_scatter(w_v, [idx], wts_v[sl])
            return carry

        lax.fori_loop(0, NP // LANES, sca, 0)
        pltpu.sync_copy(inv_v, inv_out)
        pltpu.sync_copy(w_v, w_out)


PR = UPAD // 2


@functools.partial(
    pl.kernel,
    out_type=jax.ShapeDtypeStruct((NC, UPAD, 128), jnp.float32),
    mesh=_mesh,
    scratch_types=[
        pltpu.VMEM((K,), jnp.int32),
        pltpu.VMEM((K,), jnp.int32),
        pltpu.VMEM((K,), jnp.int32),
        pltpu.VMEM((K,), jnp.int32),
        pltpu.VMEM((K, 128), jnp.float32),
        pltpu.VMEM((K, 128), jnp.float32),
        pltpu.VMEM((PR // NS, 128), jnp.float32),
        pltpu.VMEM((UPAD,), jnp.int32),
        pltpu.VMEM_SHARED((PR, 128), jnp.float32),
        pltpu.SemaphoreType.DMA,
        pltpu.SemaphoreType.DMA,
    ],
    compiler_params=pltpu.CompilerParams(needs_layout_passes=False),
)
def _sc_fine(ts_hbm, qs_hbm, t2_hbm, q2_hbm, src_hbm, dst_hbm, inv_hbm,
             out_hbm,
             src_v, dst_v, s2_v, d2_v, trows, qrows, zbuf, inv_v,
             acc, sem_t, sem_q):
    c = lax.axis_index("c")
    s = lax.axis_index("s")
    rows = PR // NS

    def zrow(i, carry):
        for d in range(8):
            zbuf[i, pl.ds(d * LANES, LANES)] = _zero16f()
        return carry

    lax.fori_loop(0, rows, zrow, 0)
    pltpu.sync_copy(inv_hbm, inv_v)

    base0 = s * (C2_CHUNKS * K)

    def run(t_hbm, q_hbm, lo):
        def chunk(g, carry):
            b = base0 + g * K
            pltpu.sync_copy(src_hbm.at[pl.ds(b, K)], src_v)
            pltpu.sync_copy(dst_hbm.at[pl.ds(b, K)], dst_v)
            for grp in range(K // LANES):
                sl = pl.ds(grp * LANES, LANES)
                sv = src_v[sl]
                dv = dst_v[sl]
                s2 = plsc.load_gather(inv_v, [sv])
                d2 = plsc.load_gather(inv_v, [dv])
                dl = dv - lo
                inb = jnp.logical_and(dl >= 0, dl < PR)
                dst_v[sl] = jnp.where(inb, dl, 0)
                s2_v[sl] = jnp.where(inb, s2, ZROW)
                d2_v[sl] = jnp.where(inb, d2, ZROW)
            ct = pltpu.async_copy(t_hbm.at[d2_v], trows, sem_t)
            cq = pltpu.async_copy(q_hbm.at[s2_v], qrows, sem_q)
            ct.wait()
            cq.wait()

            def row(i, rc):
                rsp = jnp.zeros((LANES,), jnp.int32) + i
                for d in range(8):
                    ci = d * LANES + lax.iota(jnp.int32, LANES)
                    z = (plsc.load_gather(trows, [rsp, ci])
                         + plsc.load_gather(qrows, [rsp, ci]))
                    plsc.store_scatter(trows, [rsp, ci],
                                       jnp.maximum(z, NEG * z))
                return rc

            lax.fori_loop(0, K, row, 0)
            pltpu.sync_copy(trows, acc.at[dst_v], add=True)
            return carry

        lax.fori_loop(0, C2_CHUNKS, chunk, 0)

    for p in range(2):
        lo = p * PR
        pltpu.sync_copy(zbuf, acc.at[pl.ds(s * rows, rows)])
        plsc.subcore_barrier()

        @pl.when(c == 0)
        def _skip():
            run(ts_hbm, qs_hbm, lo)

        @pl.when(c == 1)
        def _main():
            run(t2_hbm, q2_hbm, lo)

        plsc.subcore_barrier()
        pltpu.sync_copy(acc.at[pl.ds(s * rows, rows)],
                        out_hbm.at[c, pl.ds(lo + s * rows, rows)])


def kernel(x, weights, edge_index_small, m_g, m_id, W1, b1, W2, b2, Ws, bs):
    f32, i32 = jnp.float32, jnp.int32

    x_pad = jnp.zeros((NP, 128), f32).at[:N].set(x)
    W1a, W1b = W1[:128], W1[128:]
    Wsa, Wsb = Ws[:128], Ws[128:]
    Wcat = jnp.concatenate([W1a - W1b, W1b, Wsa - Wsb, Wsb], axis=1)
    bcat = jnp.concatenate(
        [b1, jnp.zeros((64,), f32), bs, jnp.zeros((128,), f32)])[None, :]

    proj = pl.pallas_call(
        _proj_body,
        out_shape=jax.ShapeDtypeStruct((NP, 384), f32),
    )(x_pad, Wcat, bcat)
    zrow = jnp.zeros((128,), f32)
    tq1 = jnp.asarray(proj[:, 0:128]).at[ZROW].set(zrow)
    ts = jnp.asarray(proj[:, 128:256]).at[ZROW].set(zrow)
    qs = jnp.asarray(proj[:, 256:384]).at[ZROW].set(zrow)

    pad1 = jnp.full((E1P - E1,), 5001, i32)
    src1 = jnp.concatenate([edge_index_small[0], pad1])
    dst1 = jnp.concatenate([edge_index_small[1], pad1])
    mid_pad = jnp.concatenate([m_id, jnp.full((NP - N,), UPAD - 8, i32)])
    w_pad = jnp.concatenate([weights[:, 0], jnp.zeros((NP - N,), f32)])

    h_parts, inv_map, w_flat = _sc_coarse(tq1, src1, dst1, mid_pad, w_pad)

    W2a, W2b = W2[:64], W2[64:]
    W2cat = jnp.concatenate([W2a - W2b, W2b], axis=1)
    b2cat = jnp.concatenate([b2, jnp.zeros((128,), f32)])[None, :]
    proj2 = pl.pallas_call(
        _proj2_body,
        out_shape=jax.ShapeDtypeStruct((NP, 256), f32),
    )(h_parts, W2cat, b2cat)
    t2 = jnp.asarray(proj2[:, 0:128]).at[ZROW].set(zrow)
    q2 = jnp.asarray(proj2[:, 128:256]).at[ZROW].set(zrow)

    pad2 = jnp.full((E2P - E2,), U, i32)
    src2 = jnp.concatenate([m_g[0], pad2])
    dst2 = jnp.concatenate([m_g[1], pad2])

    acc2 = _sc_fine(ts, qs, t2, q2, src2, dst2, inv_map)

    out = pl.pallas_call(
        _combine_body,
        out_shape=jax.ShapeDtypeStruct((U, 128), f32),
    )(acc2)
    w_up = w_flat[:U, None]
    return out, w_up
